# Initial kernel scaffold; baseline (speedup 1.0000x reference)
import jax
import jax.numpy as jnp
from jax.experimental import pallas as pl


def kernel(V0, S0, edge_index, edge_attr, params, M, obs_matrix):
    raise NotImplementedError("write your pallas kernel here")


# speedup vs baseline: 1.6298x; 1.6298x over previous
import functools

import jax
import jax.numpy as jnp
from jax import lax
from jax.experimental import pallas as pl
from jax.experimental.pallas import tpu as pltpu
from jax.experimental.pallas import tpu_sc as plsc

NV = 20000
E = 320000
D = 128
ED = 16
NF = 2

NC = 2
NTILES = 16
CHUNK = 128
E_PAD = ((E + NTILES * CHUNK - 1) // (NTILES * CHUNK)) * (NTILES * CHUNK)
EPT = E_PAD // NTILES
N_CHUNKS = EPT // CHUNK
HALF = NV // NC
DUMP = HALF
ROWS_PER_TILE = -(-(HALF + 1) // (NTILES * 8)) * 8
ACC_ROWS = ROWS_PER_TILE * NTILES


def _make_edge_agg():
    mesh = plsc.VectorSubcoreMesh(core_axis_name="c", subcore_axis_name="s")

    @functools.partial(
        pl.kernel,
        mesh=mesh,
        out_type=jax.ShapeDtypeStruct((NC * ACC_ROWS, D), jnp.float32),
        scratch_types=[
            pltpu.VMEM((CHUNK,), jnp.int32),
            pltpu.VMEM((CHUNK,), jnp.int32),
            pltpu.VMEM((CHUNK,), jnp.int32),
            pltpu.VMEM((CHUNK,), jnp.int32),
            pltpu.VMEM((CHUNK, D), jnp.float32),
            pltpu.VMEM((CHUNK, D), jnp.float32),
            pltpu.VMEM((CHUNK, D), jnp.float32),
            pltpu.VMEM_SHARED((ACC_ROWS, D), jnp.float32),
            pltpu.SemaphoreType.DMA,
            pltpu.SemaphoreType.DMA,
            pltpu.SemaphoreType.DMA,
        ],
    )
    def edge_agg(a_hbm, b_hbm, c_hbm, src_hbm, dst_hbm, out_hbm,
                 src_v, dst_v, bidx_v, loc_v, a_v, b_v, c_v, acc_sh,
                 sem0, sem1, sem2):
        cid = lax.axis_index("c")
        sid = lax.axis_index("s")
        base = cid * HALF

        zero16 = jnp.zeros((16,), jnp.float32)

        def zbody(i, carry):
            for j in range(D // 16):
                c_v[i, pl.ds(j * 16, 16)] = zero16
            return carry

        lax.fori_loop(0, CHUNK, zbody, 0)
        r0 = sid * ROWS_PER_TILE
        done = 0
        while done < ROWS_PER_TILE:
            sz = min(CHUNK, ROWS_PER_TILE - done)
            pltpu.sync_copy(c_v.at[pl.ds(0, sz)],
                            acc_sh.at[pl.ds(r0 + done, sz)])
            done += sz
        plsc.subcore_barrier()

        e0 = sid * EPT

        def chunk_body(k, carry):
            eoff = e0 + k * CHUNK
            cp_src = pltpu.async_copy(src_hbm.at[pl.ds(eoff, CHUNK)], src_v, sem0)
            cp_dst = pltpu.async_copy(dst_hbm.at[pl.ds(eoff, CHUNK)], dst_v, sem1)
            cp_c = pltpu.async_copy(c_hbm.at[pl.ds(eoff, CHUNK)], c_v, sem2)
            cp_src.wait()
            cp_a = pltpu.async_copy(a_hbm.at[src_v], a_v, sem0)
            cp_dst.wait()

            def ibody(g, carry):
                dd = dst_v[pl.ds(g * 16, 16)]
                bidx_v[pl.ds(g * 16, 16)] = jnp.minimum(
                    jnp.maximum(dd, 0), NV - 1)
                dl = dd - base
                ok = (dl >= 0) & (dl < HALF)
                loc_v[pl.ds(g * 16, 16)] = jnp.where(ok, dl, DUMP)
                return carry

            lax.fori_loop(0, CHUNK // 16, ibody, 0)
            cp_b = pltpu.async_copy(b_hbm.at[bidx_v], b_v, sem1)
            cp_c.wait()
            cp_a.wait()
            cp_b.wait()

            def mbody(e, carry):
                for j in range(D // 16):
                    s_ = pl.ds(j * 16, 16)
                    c_v[e, s_] = jnp.maximum(
                        a_v[e, s_] + b_v[e, s_] + c_v[e, s_], 0.0)
                return carry

            lax.fori_loop(0, CHUNK, mbody, 0)
            pltpu.sync_copy(c_v, acc_sh.at[loc_v], add=True)
            return carry

        lax.fori_loop(0, N_CHUNKS, chunk_body, 0)
        plsc.subcore_barrier()

        pltpu.sync_copy(acc_sh.at[pl.ds(r0, ROWS_PER_TILE)],
                        out_hbm.at[pl.ds(cid * ACC_ROWS + r0, ROWS_PER_TILE)])

    return edge_agg


_edge_agg = _make_edge_agg()


def _sc_agg(A, B, C, src, dst):
    out = _edge_agg(A, B, C, src, dst)
    return jnp.concatenate(
        [out[:HALF], out[ACC_ROWS:ACC_ROWS + HALF]], axis=0)


def _lin_body(x_ref, w_ref, b_ref, o_ref, *, act, res):
    y = jnp.dot(x_ref[...], w_ref[...], preferred_element_type=jnp.float32)
    y = y + b_ref[...]
    if act:
        y = jnp.maximum(y, 0.0)
    o_ref[...] = y


def _lin_res_body(x_ref, w_ref, b_ref, r_ref, o_ref):
    y = jnp.dot(x_ref[...], w_ref[...], preferred_element_type=jnp.float32)
    y = jnp.maximum(y + b_ref[...], 0.0)
    o_ref[...] = r_ref[...] + y


def _tc_lin(x, W, b, act=False, res=None, block_rows=1000):
    n, kdim = x.shape
    mdim = W.shape[1]
    assert n % block_rows == 0
    grid = (n // block_rows,)
    b2 = b.reshape(1, mdim)
    in_specs = [
        pl.BlockSpec((block_rows, kdim), lambda i: (i, 0)),
        pl.BlockSpec((kdim, mdim), lambda i: (0, 0)),
        pl.BlockSpec((1, mdim), lambda i: (0, 0)),
    ]
    args = [x, W, b2]
    if res is not None:
        in_specs.append(pl.BlockSpec((block_rows, mdim), lambda i: (i, 0)))
        args.append(res)
        body = _lin_res_body
    else:
        body = functools.partial(_lin_body, act=act, res=None)
    return pl.pallas_call(
        body,
        grid=grid,
        in_specs=in_specs,
        out_specs=pl.BlockSpec((block_rows, mdim), lambda i: (i, 0)),
        out_shape=jax.ShapeDtypeStruct((n, mdim), jnp.float32),
    )(*args)


def kernel(V0, S0, edge_index, edge_attr, params, M, obs_matrix):
    p = params
    WmVS, bmVS = p['fVS_msg']
    WuVS, buVS = p['fVS_upd']
    WmSV, bmSV = p['fSV_msg']
    WuSV, buSV = p['fSV_upd']
    Wsd, bsd = p['dStodV']
    Wds, bds = p['dVtodS']

    src = edge_index[0].astype(jnp.int32)
    dst = edge_index[1].astype(jnp.int32)
    pad_n = E_PAD - E
    big = jnp.full((pad_n,), 1 << 30, dtype=jnp.int32)
    zer = jnp.zeros((pad_n,), dtype=jnp.int32)
    src_f = jnp.concatenate([src, zer])
    dst_f = jnp.concatenate([dst, big])
    src_r = jnp.concatenate([dst, zer])
    dst_r = jnp.concatenate([src, big])

    ea_pad = jnp.concatenate(
        [edge_attr, jnp.zeros((pad_n, ED), jnp.float32)], axis=0)
    C_VS = _tc_lin(ea_pad, WmVS[2 * D:], jnp.zeros((D,), jnp.float32),
                   block_rows=2048)
    C_SV = _tc_lin(ea_pad, WmSV[2 * D:], jnp.zeros((D,), jnp.float32),
                   block_rows=2048)

    V = _tc_lin(V0, *p['embed_V'])
    S = _tc_lin(S0, *p['embed_S'])

    zb = jnp.zeros((D,), jnp.float32)
    for _ in range(NF):
        S_proj = _tc_lin(S, Wsd, bsd)
        A1 = _tc_lin(V, WmVS[:D], zb)
        B1 = _tc_lin(S_proj, WmVS[D:2 * D], bmVS)
        agg1 = _sc_agg(A1, B1, C_VS, src_f, dst_f)
        S_out = _tc_lin(agg1, WuVS, buVS, res=S_proj)
        S = _tc_lin(S_out, Wds, bds)
        V_proj = _tc_lin(V, Wds, bds)
        A2 = _tc_lin(S, WmSV[:D], zb)
        B2 = _tc_lin(V_proj, WmSV[D:2 * D], bmSV)
        agg2 = _sc_agg(A2, B2, C_SV, src_r, dst_r)
        V_out = _tc_lin(agg2, WuSV, buSV, res=V_proj)
        V = _tc_lin(V_out, Wsd, bsd)

    P = _tc_lin(V, *p['extract_V']).reshape(-1, 3, 4)
    X = _tc_lin(S, *p['extract_S'])
    X = jnp.concatenate([X, jnp.ones((X.shape[0], 1), X.dtype)], axis=1)
    return (P, X)


# speedup vs baseline: 2.0678x; 1.2687x over previous
import functools

import jax
import jax.numpy as jnp
from jax import lax
from jax.experimental import pallas as pl
from jax.experimental.pallas import tpu as pltpu
from jax.experimental.pallas import tpu_sc as plsc

NV = 20000
E = 320000
D = 128
ED = 16
NF = 2

NC = 2
NTILES = 16
CHUNK = 64
E_PAD = ((E + 2 * NTILES * CHUNK - 1) // (2 * NTILES * CHUNK)) * (2 * NTILES * CHUNK)
EPT = E_PAD // NTILES
N_CHUNKS = EPT // CHUNK
PAIRS = N_CHUNKS // 2
HALF = NV // NC
DUMP = HALF
ROWS_PER_TILE = -(-(HALF + 1) // (NTILES * 8)) * 8
ACC_ROWS = ROWS_PER_TILE * NTILES


def _make_edge_agg():
    mesh = plsc.VectorSubcoreMesh(core_axis_name="c", subcore_axis_name="s")

    nbuf = 2
    scratch = []
    for _ in range(nbuf):
        scratch += [
            pltpu.VMEM((CHUNK,), jnp.int32),
            pltpu.VMEM((CHUNK,), jnp.int32),
            pltpu.VMEM((CHUNK,), jnp.int32),
            pltpu.VMEM((CHUNK,), jnp.int32),
            pltpu.VMEM((CHUNK, D), jnp.float32),
            pltpu.VMEM((CHUNK, D), jnp.float32),
            pltpu.VMEM((CHUNK, D), jnp.float32),
        ]
    scratch.append(pltpu.VMEM_SHARED((ACC_ROWS, D), jnp.float32))
    scratch += [pltpu.SemaphoreType.DMA] * (6 * nbuf)

    @functools.partial(
        pl.kernel,
        mesh=mesh,
        out_type=jax.ShapeDtypeStruct((NC * ACC_ROWS, D), jnp.float32),
        scratch_types=scratch,
    )
    def edge_agg(a_hbm, b_hbm, c_hbm, src_hbm, dst_hbm, out_hbm, *rest):
        bufs = [rest[7 * i:7 * (i + 1)] for i in range(nbuf)]
        acc_sh = rest[7 * nbuf]
        sems = rest[7 * nbuf + 1:]
        s_src = sems[0:2]
        s_dst = sems[2:4]
        s_c = sems[4:6]
        s_a = sems[6:8]
        s_b = sems[8:10]
        s_scat = sems[10:12]
        src_v = [bufs[i][0] for i in range(nbuf)]
        dst_v = [bufs[i][1] for i in range(nbuf)]
        bidx_v = [bufs[i][2] for i in range(nbuf)]
        loc_v = [bufs[i][3] for i in range(nbuf)]
        a_v = [bufs[i][4] for i in range(nbuf)]
        b_v = [bufs[i][5] for i in range(nbuf)]
        c_v = [bufs[i][6] for i in range(nbuf)]

        cid = lax.axis_index("c")
        sid = lax.axis_index("s")
        base = cid * HALF

        zero16 = jnp.zeros((16,), jnp.float32)

        def zbody(i, carry):
            for j in range(D // 16):
                c_v[0][i, pl.ds(j * 16, 16)] = zero16
            return carry

        lax.fori_loop(0, CHUNK, zbody, 0)
        r0 = sid * ROWS_PER_TILE
        done = 0
        while done < ROWS_PER_TILE:
            sz = min(CHUNK, ROWS_PER_TILE - done)
            pltpu.sync_copy(c_v[0].at[pl.ds(0, sz)],
                            acc_sh.at[pl.ds(r0 + done, sz)])
            done += sz
        plsc.subcore_barrier()

        e0 = sid * EPT

        def fire(b, ch, drain):
            eoff = e0 + ch * CHUNK
            if drain:
                pltpu.make_async_copy(
                    c_v[b], acc_sh.at[loc_v[b]], s_scat[b]).wait()
            d_src = pltpu.async_copy(
                src_hbm.at[pl.ds(eoff, CHUNK)], src_v[b], s_src[b])
            d_dst = pltpu.async_copy(
                dst_hbm.at[pl.ds(eoff, CHUNK)], dst_v[b], s_dst[b])
            pltpu.async_copy(c_hbm.at[pl.ds(eoff, CHUNK)], c_v[b], s_c[b])
            d_src.wait()
            pltpu.async_copy(a_hbm.at[src_v[b]], a_v[b], s_a[b])
            d_dst.wait()

            def ibody(g, carry):
                dd = dst_v[b][pl.ds(g * 16, 16)]
                bidx_v[b][pl.ds(g * 16, 16)] = jnp.minimum(
                    jnp.maximum(dd, 0), NV - 1)
                dl = dd - base
                ok = (dl >= 0) & (dl < HALF)
                loc_v[b][pl.ds(g * 16, 16)] = jnp.where(ok, dl, DUMP)
                return carry

            lax.fori_loop(0, CHUNK // 16, ibody, 0)
            pltpu.async_copy(b_hbm.at[bidx_v[b]], b_v[b], s_b[b])

        def finish(b):
            pltpu.make_async_copy(a_hbm.at[src_v[b]], a_v[b], s_a[b]).wait()
            pltpu.make_async_copy(b_hbm.at[bidx_v[b]], b_v[b], s_b[b]).wait()
            pltpu.make_async_copy(
                c_hbm.at[pl.ds(0, CHUNK)], c_v[b], s_c[b]).wait()

            def mbody(e, carry):
                for j in range(D // 16):
                    s_ = pl.ds(j * 16, 16)
                    c_v[b][e, s_] = jnp.maximum(
                        a_v[b][e, s_] + b_v[b][e, s_] + c_v[b][e, s_], 0.0)
                return carry

            lax.fori_loop(0, CHUNK, mbody, 0)
            pltpu.async_copy(c_v[b], acc_sh.at[loc_v[b]], s_scat[b],
                             add=True)

        fire(0, 0, False)
        fire(1, 1, False)

        def pair_body(p, carry):
            finish(0)
            fire(0, 2 * p + 2, True)
            finish(1)
            fire(1, 2 * p + 3, True)
            return carry

        lax.fori_loop(0, PAIRS - 1, pair_body, 0)
        finish(0)
        finish(1)
        for b in range(nbuf):
            pltpu.make_async_copy(
                c_v[b], acc_sh.at[loc_v[b]], s_scat[b]).wait()
        plsc.subcore_barrier()

        pltpu.sync_copy(acc_sh.at[pl.ds(r0, ROWS_PER_TILE)],
                        out_hbm.at[pl.ds(cid * ACC_ROWS + r0, ROWS_PER_TILE)])

    return edge_agg


_edge_agg = _make_edge_agg()


def _sc_agg(A, B, C, src, dst):
    out = _edge_agg(A, B, C, src, dst)
    return jnp.concatenate(
        [out[:HALF], out[ACC_ROWS:ACC_ROWS + HALF]], axis=0)


def _lin_body(x_ref, w_ref, b_ref, o_ref, *, act, res):
    y = jnp.dot(x_ref[...], w_ref[...], preferred_element_type=jnp.float32)
    y = y + b_ref[...]
    if act:
        y = jnp.maximum(y, 0.0)
    o_ref[...] = y


def _lin_res_body(x_ref, w_ref, b_ref, r_ref, o_ref):
    y = jnp.dot(x_ref[...], w_ref[...], preferred_element_type=jnp.float32)
    y = jnp.maximum(y + b_ref[...], 0.0)
    o_ref[...] = r_ref[...] + y


def _tc_lin(x, W, b, act=False, res=None, block_rows=1000):
    n, kdim = x.shape
    mdim = W.shape[1]
    assert n % block_rows == 0
    grid = (n // block_rows,)
    b2 = b.reshape(1, mdim)
    in_specs = [
        pl.BlockSpec((block_rows, kdim), lambda i: (i, 0)),
        pl.BlockSpec((kdim, mdim), lambda i: (0, 0)),
        pl.BlockSpec((1, mdim), lambda i: (0, 0)),
    ]
    args = [x, W, b2]
    if res is not None:
        in_specs.append(pl.BlockSpec((block_rows, mdim), lambda i: (i, 0)))
        args.append(res)
        body = _lin_res_body
    else:
        body = functools.partial(_lin_body, act=act, res=None)
    return pl.pallas_call(
        body,
        grid=grid,
        in_specs=in_specs,
        out_specs=pl.BlockSpec((block_rows, mdim), lambda i: (i, 0)),
        out_shape=jax.ShapeDtypeStruct((n, mdim), jnp.float32),
    )(*args)


def kernel(V0, S0, edge_index, edge_attr, params, M, obs_matrix):
    p = params
    WmVS, bmVS = p['fVS_msg']
    WuVS, buVS = p['fVS_upd']
    WmSV, bmSV = p['fSV_msg']
    WuSV, buSV = p['fSV_upd']
    Wsd, bsd = p['dStodV']
    Wds, bds = p['dVtodS']

    src = edge_index[0].astype(jnp.int32)
    dst = edge_index[1].astype(jnp.int32)
    pad_n = E_PAD - E
    big = jnp.full((pad_n,), 1 << 30, dtype=jnp.int32)
    zer = jnp.zeros((pad_n,), dtype=jnp.int32)
    src_f = jnp.concatenate([src, zer])
    dst_f = jnp.concatenate([dst, big])
    src_r = jnp.concatenate([dst, zer])
    dst_r = jnp.concatenate([src, big])

    ea_pad = jnp.concatenate(
        [edge_attr, jnp.zeros((pad_n, ED), jnp.float32)], axis=0)
    C_VS = _tc_lin(ea_pad, WmVS[2 * D:], jnp.zeros((D,), jnp.float32),
                   block_rows=2048)
    C_SV = _tc_lin(ea_pad, WmSV[2 * D:], jnp.zeros((D,), jnp.float32),
                   block_rows=2048)

    V = _tc_lin(V0, *p['embed_V'])
    S = _tc_lin(S0, *p['embed_S'])

    zb = jnp.zeros((D,), jnp.float32)
    for _ in range(NF):
        S_proj = _tc_lin(S, Wsd, bsd)
        A1 = _tc_lin(V, WmVS[:D], zb)
        B1 = _tc_lin(S_proj, WmVS[D:2 * D], bmVS)
        agg1 = _sc_agg(A1, B1, C_VS, src_f, dst_f)
        S_out = _tc_lin(agg1, WuVS, buVS, res=S_proj)
        S = _tc_lin(S_out, Wds, bds)
        V_proj = _tc_lin(V, Wds, bds)
        A2 = _tc_lin(S, WmSV[:D], zb)
        B2 = _tc_lin(V_proj, WmSV[D:2 * D], bmSV)
        agg2 = _sc_agg(A2, B2, C_SV, src_r, dst_r)
        V_out = _tc_lin(agg2, WuSV, buSV, res=V_proj)
        V = _tc_lin(V_out, Wsd, bsd)

    P = _tc_lin(V, *p['extract_V']).reshape(-1, 3, 4)
    X = _tc_lin(S, *p['extract_S'])
    X = jnp.concatenate([X, jnp.ones((X.shape[0], 1), X.dtype)], axis=1)
    return (P, X)
